# baseline (speedup 1.0000x reference)
import functools

import jax
import jax.numpy as jnp
import numpy as np
from jax import lax
from jax.experimental import pallas as pl
from jax.experimental.pallas import tpu as pltpu
from jax.experimental.pallas import tpu_sc as plsc

B = 16384
D = 32
NUM_CORES = 2
NUM_SUBCORES = 16
LANES = 16
NW = NUM_CORES * NUM_SUBCORES
BPW = B // NW
CHUNK = 128
NCHUNK = BPW // CHUNK

def _hsum_all_lanes(v):
    lane = lax.iota(jnp.int32, LANES)
    for k in (8, 4, 2, 1):
        rot = lax.bitwise_and(lane + k, LANES - 1)
        v = v + jnp.take_along_axis(v, rot, axis=0)
    return v


def _rsqrt_newton(t):
    i = lax.bitcast_convert_type(t, jnp.int32)
    y = lax.bitcast_convert_type(
        jnp.int32(0x5F3759DF) - lax.shift_right_logical(i, 1), jnp.float32
    )
    ht = t * jnp.float32(0.5)
    for _ in range(3):
        y = y * (jnp.float32(1.5) - ht * y * y)
    return y


def _sc_body(idx_hbm, table_hbm, out_hbm, idx_v, rows_v, sem):
    wid = lax.axis_index("s") * NUM_CORES + lax.axis_index("c")
    base = wid * BPW

    for j in range(NCHUNK):
        pltpu.sync_copy(idx_hbm.at[pl.ds(base + j * CHUNK, CHUNK)], idx_v.at[j])

    copies = [
        pltpu.async_copy(
            table_hbm.at[idx_v.at[j]],
            rows_v.at[pl.ds(j * CHUNK, CHUNK)],
            sem,
        )
        for j in range(NCHUNK)
    ]
    for c in copies:
        c.wait()

    def row_fn(r, carry):
        v0 = rows_v[r, pl.ds(0, LANES)]
        v1 = rows_v[r, pl.ds(LANES, LANES)]
        t = _hsum_all_lanes(v0 * v0 + v1 * v1)
        inv = _rsqrt_newton(t)
        rows_v[r, pl.ds(0, LANES)] = v0 * inv
        rows_v[r, pl.ds(LANES, LANES)] = v1 * inv
        return carry

    lax.fori_loop(0, BPW, row_fn, 0, unroll=2)

    pltpu.sync_copy(rows_v, out_hbm.at[pl.ds(base, BPW)])


def kernel(class_indices, embedding_weight):
    if class_indices.ndim > 1:
        class_indices = class_indices.squeeze(-1)
    idx = class_indices.astype(jnp.int32)

    mesh = plsc.VectorSubcoreMesh(
        core_axis_name="c",
        subcore_axis_name="s",
        num_cores=NUM_CORES,
        num_subcores=NUM_SUBCORES,
    )
    run = pl.kernel(
        _sc_body,
        out_type=jax.ShapeDtypeStruct((B, D), jnp.float32),
        mesh=mesh,
        scratch_types=[
            pltpu.VMEM((NCHUNK, CHUNK), jnp.int32),
            pltpu.VMEM((BPW, D), jnp.float32),
            pltpu.SemaphoreType.DMA,
        ],
        compiler_params=pltpu.CompilerParams(use_tc_tiling_on_sc=False),
    )
    return run(idx, embedding_weight)
